# baseline (speedup 1.0000x reference)
import functools

import jax
import jax.numpy as jnp
from jax import lax
from jax.experimental import pallas as pl
from jax.experimental.pallas import tpu as pltpu
from jax.experimental.pallas import tpu_sc as plsc

N, E, D, DE = 10000, 320000, 128, 16
NC, NS = 2, 16
NW = NC * NS
CHUNK = 64
NCHUNK_A = 36
EA = NW * NCHUNK_A * CHUNK
EB = E - EA
NCHUNK_B = 120
NTAIL_B = EB // CHUNK - NW * NCHUNK_B
NPAD = 10240
ROWS_PER_SUB = NPAD // NS


def _proj_body(ea_ref, w_ref, b_ref, out_ref):
    out_ref[...] = (
        jnp.dot(ea_ref[...], w_ref[...], preferred_element_type=jnp.float32)
        + b_ref[...]
    )


def _edge_proj(edge_attr, w, b, ne, be):
    return pl.pallas_call(
        _proj_body,
        grid=(ne // be,),
        in_specs=[
            pl.BlockSpec((be, DE), lambda i: (i, 0)),
            pl.BlockSpec((DE, D), lambda i: (0, 0)),
            pl.BlockSpec((1, D), lambda i: (0, 0)),
        ],
        out_specs=pl.BlockSpec((be, D), lambda i: (i, 0)),
        out_shape=jax.ShapeDtypeStruct((ne, D), jnp.float32),
    )(edge_attr, w, b.reshape(1, D))


def _sc_aggregate(x, src, dst, eproj, init, nchunk, ntail):
    mesh = plsc.VectorSubcoreMesh(core_axis_name="c", subcore_axis_name="s")

    @functools.partial(
        pl.kernel,
        mesh=mesh,
        out_type=jax.ShapeDtypeStruct((NC, NPAD, D), jnp.float32),
        scratch_types=(
            [pltpu.VMEM((CHUNK,), jnp.int32)] * 2
            + [pltpu.VMEM((CHUNK,), jnp.int32)] * 3
            + [pltpu.VMEM((CHUNK, D), jnp.float32)] * 2
            + [pltpu.VMEM((CHUNK, D), jnp.float32)] * 3
            + [pltpu.VMEM_SHARED((NPAD, D), jnp.float32)]
            + [pltpu.SemaphoreType.DMA] * 13
        ),
    )
    def k(x_hbm, src_hbm, dst_hbm, ep_hbm, init_hbm, out_hbm,
          sv0, sv1, dv0, dv1, dv2, xg0, xg1, ev0, ev1, ev2, aggsh,
          gsem0, gsem1, esem0, esem1, esem2, ssem0, ssem1, ssem2,
          isem0, isem1, dsem0, dsem1, dsem2):
        sv = (sv0, sv1)
        dv = (dv0, dv1, dv2)
        xg = (xg0, xg1)
        ev = (ev0, ev1, ev2)
        gsem = (gsem0, gsem1)
        esem = (esem0, esem1, esem2)
        ssem = (ssem0, ssem1, ssem2)
        isem = (isem0, isem1)
        dsem = (dsem0, dsem1, dsem2)
        cid = lax.axis_index("c")
        sid = lax.axis_index("s")
        wid = sid * NC + cid
        pltpu.sync_copy(
            init_hbm.at[cid, pl.ds(sid * ROWS_PER_SUB, ROWS_PER_SUB)],
            aggsh.at[pl.ds(sid * ROWS_PER_SUB, ROWS_PER_SUB)],
        )
        plsc.subcore_barrier()
        ebase = wid * nchunk * CHUNK

        def src_cp(c, s2):
            return pltpu.make_async_copy(
                src_hbm.at[pl.ds(ebase + c * CHUNK, CHUNK)],
                sv[s2], isem[s2])

        def dst_cp(c, s3):
            return pltpu.make_async_copy(
                dst_hbm.at[pl.ds(ebase + c * CHUNK, CHUNK)],
                dv[s3], dsem[s3])

        def gather_cp(s2):
            return pltpu.make_async_copy(
                x_hbm.at[sv[s2]], xg[s2], gsem[s2])

        def ep_cp(c, s3):
            return pltpu.make_async_copy(
                ep_hbm.at[pl.ds(ebase + c * CHUNK, CHUNK)],
                ev[s3], esem[s3])

        def scat_cp(s3):
            return pltpu.make_async_copy(
                ev[s3], aggsh.at[dv[s3]], ssem[s3])

        def compute(s2, s3):
            xb, eb = xg[s2], ev[s3]

            @pl.loop(0, CHUNK)
            def _(i):
                for j in range(D // 16):
                    sl = pl.ds(j * 16, 16)
                    eb[i, sl] = jnp.maximum(eb[i, sl] + xb[i, sl], 0.0)

        src_cp(0, 0).start()
        dst_cp(0, 0).start()
        src_cp(1, 1).start()
        src_cp(0, 0).wait()
        gather_cp(0).start()
        ep_cp(0, 0).start()
        dst_cp(1, 1).start()

        @pl.loop(0, nchunk, step=6)
        def _(g):
            for u in range(6):
                c = g + u
                s2, s3 = u % 2, u % 3
                n2, n3 = (u + 1) % 2, (u + 1) % 3
                p3 = (u - 1) % 3

                gather_cp(s2).wait()
                ep_cp(c, s3).wait()

                @pl.when(c + 1 < nchunk)
                def _():
                    src_cp(c + 1, n2).wait()
                    gather_cp(n2).start()
                    ep_cp(c + 1, n3).start()

                    @pl.when(c >= 1)
                    def _():
                        dst_cp(c + 1, n3).start()

                @pl.when(c + 2 < nchunk)
                def _():
                    src_cp(c + 2, s2).start()

                compute(s2, s3)

                dst_cp(c, s3).wait()

                @pl.when(c >= 1)
                def _():
                    scat_cp(p3).wait()

                scat_cp(s3).start(add=True)

        scat_cp((nchunk - 1) % 3).wait()

        @pl.when(wid < ntail)
        def _():
            toff = (NW * nchunk + wid) * CHUNK

            def t_src():
                return pltpu.make_async_copy(
                    src_hbm.at[pl.ds(toff, CHUNK)], sv[0], isem[0])

            def t_dst():
                return pltpu.make_async_copy(
                    dst_hbm.at[pl.ds(toff, CHUNK)], dv[0], dsem[0])

            def t_ep():
                return pltpu.make_async_copy(
                    ep_hbm.at[pl.ds(toff, CHUNK)], ev[0], esem[0])

            t_src().start()
            t_dst().start()
            t_src().wait()
            gather_cp(0).start()
            t_ep().start()
            gather_cp(0).wait()
            t_ep().wait()
            compute(0, 0)
            t_dst().wait()
            scat_cp(0).start(add=True)
            scat_cp(0).wait()

        plsc.subcore_barrier()
        pltpu.sync_copy(
            aggsh.at[pl.ds(sid * ROWS_PER_SUB, ROWS_PER_SUB)],
            out_hbm.at[cid, pl.ds(sid * ROWS_PER_SUB, ROWS_PER_SUB)],
        )

    return k(x, src, dst, eproj, init)


def _mlp_norm_body(x_ref, p_ref, w1_ref, b1_ref, w2_ref, b2_ref,
                   gw_ref, gb_ref, gs_ref, out_ref):
    h = x_ref[...] + p_ref[0] + p_ref[1]
    a = jnp.maximum(
        jnp.dot(h, w1_ref[...], preferred_element_type=jnp.float32)
        + b1_ref[...], 0.0)
    t = (jnp.dot(a, w2_ref[...], preferred_element_type=jnp.float32)
         + b2_ref[...])
    m = jnp.mean(t, axis=0, keepdims=True)
    c = t - gs_ref[...] * m
    v = jnp.mean(c * c, axis=0, keepdims=True)
    out_ref[...] = jnp.maximum(
        gw_ref[...] * c * lax.rsqrt(v + 1e-5) + gb_ref[...], 0.0)


def _mlp_norm(x, partials, W1, b1, W2, b2, gn_weight, gn_bias, gn_mean_scale):
    return pl.pallas_call(
        _mlp_norm_body,
        out_shape=jax.ShapeDtypeStruct((N, D), jnp.float32),
    )(x, partials, W1, b1.reshape(1, D), W2, b2.reshape(1, D),
      gn_weight.reshape(1, D), gn_bias.reshape(1, D),
      gn_mean_scale.reshape(1, D))


def kernel(x, edge_index, edge_attr, lin_e_W, lin_e_b, W1, b1, W2, b2,
           gn_weight, gn_bias, gn_mean_scale):
    src = edge_index[0]
    dst = edge_index[1]
    eprojA = _edge_proj(edge_attr[:EA], lin_e_W, lin_e_b, EA, 4608)
    eprojB = _edge_proj(edge_attr[EA:], lin_e_W, lin_e_b, EB, 7696)
    zeros = jnp.zeros((NC, NPAD, D), jnp.float32)
    partA = _sc_aggregate(x, src[:EA], dst[:EA], eprojA, zeros,
                          NCHUNK_A, 0)
    partials = _sc_aggregate(x, src[EA:], dst[EA:], eprojB, partA,
                             NCHUNK_B, NTAIL_B)
    return _mlp_norm(x, partials[:, :N, :], W1, b1, W2, b2,
                     gn_weight, gn_bias, gn_mean_scale)


# speedup vs baseline: 1.0179x; 1.0179x over previous
import functools

import jax
import jax.numpy as jnp
from jax import lax
from jax.experimental import pallas as pl
from jax.experimental.pallas import tpu as pltpu
from jax.experimental.pallas import tpu_sc as plsc

N, E, D, DE = 10000, 320000, 128, 16
NC, NS = 2, 16
NW = NC * NS
CHUNK = 64
NCHUNK_A = 66
EA = NW * NCHUNK_A * CHUNK
EB = E - EA
NCHUNK_B = 90
NTAIL_B = EB // CHUNK - NW * NCHUNK_B
NPAD = 10240
ROWS_PER_SUB = NPAD // NS


def _proj_body(ea_ref, w_ref, b_ref, out_ref):
    out_ref[...] = (
        jnp.dot(ea_ref[...], w_ref[...], preferred_element_type=jnp.float32)
        + b_ref[...]
    )


def _edge_proj(edge_attr, w, b, ne, be):
    return pl.pallas_call(
        _proj_body,
        grid=(ne // be,),
        in_specs=[
            pl.BlockSpec((be, DE), lambda i: (i, 0)),
            pl.BlockSpec((DE, D), lambda i: (0, 0)),
            pl.BlockSpec((1, D), lambda i: (0, 0)),
        ],
        out_specs=pl.BlockSpec((be, D), lambda i: (i, 0)),
        out_shape=jax.ShapeDtypeStruct((ne, D), jnp.float32),
    )(edge_attr, w, b.reshape(1, D))


def _sc_aggregate(x, src, dst, eproj, init, nchunk, ntail):
    mesh = plsc.VectorSubcoreMesh(core_axis_name="c", subcore_axis_name="s")

    @functools.partial(
        pl.kernel,
        mesh=mesh,
        out_type=jax.ShapeDtypeStruct((NC, NPAD, D), jnp.float32),
        scratch_types=(
            [pltpu.VMEM((CHUNK,), jnp.int32)] * 2
            + [pltpu.VMEM((CHUNK,), jnp.int32)] * 3
            + [pltpu.VMEM((CHUNK, D), jnp.float32)] * 2
            + [pltpu.VMEM((CHUNK, D), jnp.float32)] * 3
            + [pltpu.VMEM_SHARED((NPAD, D), jnp.float32)]
            + [pltpu.SemaphoreType.DMA] * 13
        ),
    )
    def k(x_hbm, src_hbm, dst_hbm, ep_hbm, init_hbm, out_hbm,
          sv0, sv1, dv0, dv1, dv2, xg0, xg1, ev0, ev1, ev2, aggsh,
          gsem0, gsem1, esem0, esem1, esem2, ssem0, ssem1, ssem2,
          isem0, isem1, dsem0, dsem1, dsem2):
        sv = (sv0, sv1)
        dv = (dv0, dv1, dv2)
        xg = (xg0, xg1)
        ev = (ev0, ev1, ev2)
        gsem = (gsem0, gsem1)
        esem = (esem0, esem1, esem2)
        ssem = (ssem0, ssem1, ssem2)
        isem = (isem0, isem1)
        dsem = (dsem0, dsem1, dsem2)
        cid = lax.axis_index("c")
        sid = lax.axis_index("s")
        wid = sid * NC + cid
        pltpu.sync_copy(
            init_hbm.at[cid, pl.ds(sid * ROWS_PER_SUB, ROWS_PER_SUB)],
            aggsh.at[pl.ds(sid * ROWS_PER_SUB, ROWS_PER_SUB)],
        )
        plsc.subcore_barrier()
        ebase = wid * nchunk * CHUNK

        def src_cp(c, s2):
            return pltpu.make_async_copy(
                src_hbm.at[pl.ds(ebase + c * CHUNK, CHUNK)],
                sv[s2], isem[s2])

        def dst_cp(c, s3):
            return pltpu.make_async_copy(
                dst_hbm.at[pl.ds(ebase + c * CHUNK, CHUNK)],
                dv[s3], dsem[s3])

        def gather_cp(s2):
            return pltpu.make_async_copy(
                x_hbm.at[sv[s2]], xg[s2], gsem[s2])

        def ep_cp(c, s3):
            return pltpu.make_async_copy(
                ep_hbm.at[pl.ds(ebase + c * CHUNK, CHUNK)],
                ev[s3], esem[s3])

        def scat_cp(s3):
            return pltpu.make_async_copy(
                ev[s3], aggsh.at[dv[s3]], ssem[s3])

        def compute(s2, s3):
            xb, eb = xg[s2], ev[s3]

            @pl.loop(0, CHUNK)
            def _(i):
                for j in range(D // 16):
                    sl = pl.ds(j * 16, 16)
                    eb[i, sl] = jnp.maximum(eb[i, sl] + xb[i, sl], 0.0)

        src_cp(0, 0).start()
        dst_cp(0, 0).start()
        src_cp(1, 1).start()
        src_cp(0, 0).wait()
        gather_cp(0).start()
        ep_cp(0, 0).start()
        dst_cp(1, 1).start()

        @pl.loop(0, nchunk, step=6)
        def _(g):
            for u in range(6):
                c = g + u
                s2, s3 = u % 2, u % 3
                n2, n3 = (u + 1) % 2, (u + 1) % 3
                p3 = (u - 1) % 3

                gather_cp(s2).wait()
                ep_cp(c, s3).wait()

                @pl.when(c + 1 < nchunk)
                def _():
                    src_cp(c + 1, n2).wait()
                    gather_cp(n2).start()
                    ep_cp(c + 1, n3).start()

                    @pl.when(c >= 1)
                    def _():
                        dst_cp(c + 1, n3).start()

                @pl.when(c + 2 < nchunk)
                def _():
                    src_cp(c + 2, s2).start()

                compute(s2, s3)

                dst_cp(c, s3).wait()

                @pl.when(c >= 1)
                def _():
                    scat_cp(p3).wait()

                scat_cp(s3).start(add=True)

        scat_cp((nchunk - 1) % 3).wait()

        @pl.when(wid < ntail)
        def _():
            toff = (NW * nchunk + wid) * CHUNK

            def t_src():
                return pltpu.make_async_copy(
                    src_hbm.at[pl.ds(toff, CHUNK)], sv[0], isem[0])

            def t_dst():
                return pltpu.make_async_copy(
                    dst_hbm.at[pl.ds(toff, CHUNK)], dv[0], dsem[0])

            def t_ep():
                return pltpu.make_async_copy(
                    ep_hbm.at[pl.ds(toff, CHUNK)], ev[0], esem[0])

            t_src().start()
            t_dst().start()
            t_src().wait()
            gather_cp(0).start()
            t_ep().start()
            gather_cp(0).wait()
            t_ep().wait()
            compute(0, 0)
            t_dst().wait()
            scat_cp(0).start(add=True)
            scat_cp(0).wait()

        plsc.subcore_barrier()
        pltpu.sync_copy(
            aggsh.at[pl.ds(sid * ROWS_PER_SUB, ROWS_PER_SUB)],
            out_hbm.at[cid, pl.ds(sid * ROWS_PER_SUB, ROWS_PER_SUB)],
        )

    return k(x, src, dst, eproj, init)


def _mlp_norm_body(x_ref, p_ref, w1_ref, b1_ref, w2_ref, b2_ref,
                   gw_ref, gb_ref, gs_ref, out_ref):
    h = x_ref[...] + p_ref[0] + p_ref[1]
    a = jnp.maximum(
        jnp.dot(h, w1_ref[...], preferred_element_type=jnp.float32)
        + b1_ref[...], 0.0)
    t = (jnp.dot(a, w2_ref[...], preferred_element_type=jnp.float32)
         + b2_ref[...])
    m = jnp.mean(t, axis=0, keepdims=True)
    c = t - gs_ref[...] * m
    v = jnp.mean(c * c, axis=0, keepdims=True)
    out_ref[...] = jnp.maximum(
        gw_ref[...] * c * lax.rsqrt(v + 1e-5) + gb_ref[...], 0.0)


def _mlp_norm(x, partials, W1, b1, W2, b2, gn_weight, gn_bias, gn_mean_scale):
    return pl.pallas_call(
        _mlp_norm_body,
        out_shape=jax.ShapeDtypeStruct((N, D), jnp.float32),
    )(x, partials, W1, b1.reshape(1, D), W2, b2.reshape(1, D),
      gn_weight.reshape(1, D), gn_bias.reshape(1, D),
      gn_mean_scale.reshape(1, D))


def kernel(x, edge_index, edge_attr, lin_e_W, lin_e_b, W1, b1, W2, b2,
           gn_weight, gn_bias, gn_mean_scale):
    src = edge_index[0]
    dst = edge_index[1]
    eprojA = _edge_proj(edge_attr[:EA], lin_e_W, lin_e_b, EA, 8448)
    eprojB = _edge_proj(edge_attr[EA:], lin_e_W, lin_e_b, EB, 5776)
    zeros = jnp.zeros((NC, NPAD, D), jnp.float32)
    partA = _sc_aggregate(x, src[:EA], dst[:EA], eprojA, zeros,
                          NCHUNK_A, 0)
    partials = _sc_aggregate(x, src[EA:], dst[EA:], eprojB, partA,
                             NCHUNK_B, NTAIL_B)
    return _mlp_norm(x, partials[:, :N, :], W1, b1, W2, b2,
                     gn_weight, gn_bias, gn_mean_scale)
